# baseline (speedup 1.0000x reference)
import jax
import jax.numpy as jnp
import numpy as np
from jax import lax
from jax.experimental import pallas as pl
from jax.experimental.pallas import tpu as pltpu
from jax.experimental.pallas import tpu_sc as plsc

_CAT_DIMS = [100000] * 26
_KC = len(_CAT_DIMS)
_D = 16
_BATCH = 16384
_NC, _NS = 2, 16
_NW = _NC * _NS
_BPW = _BATCH // _NW
_SB = 64
_NSUB = _BPW // _SB
_LK = _SB * _KC
_GW = 128
_NG = _LK // _GW
_OFF_PERIOD = 208
_BIAS_PERIOD = _KC * _D

_OFFSETS = np.cumsum([0] + _CAT_DIMS[:-1]).astype(np.int32)
_OFF_PATTERN = np.tile(_OFFSETS, _OFF_PERIOD // _KC)


def _sc_body(x_hbm, e_hbm, bias_hbm, off_hbm, out_hbm,
             idx_v, rows_v, bias_sm, off_sm, gsem, osem):
    wid = lax.axis_index("s") * _NC + lax.axis_index("c")
    pltpu.sync_copy(bias_hbm, bias_sm)
    pltpu.sync_copy(off_hbm, off_sm)
    base0 = wid * (_BPW * _KC)
    batch0 = wid * _BPW

    def stage_and_fire(t, buf):
        base = base0 + t * _LK
        pltpu.sync_copy(x_hbm.at[pl.ds(base, _LK)], idx_v.at[buf])

        def off_body(j, c):
            sl = pl.ds(j * 16, 16)
            idx_v[buf, sl] = idx_v[buf, sl] + off_sm[pl.ds(c, 16)]
            c = c + 16
            return jnp.where(c == _OFF_PERIOD, 0, c)
        lax.fori_loop(0, _LK // 16, off_body, jnp.int32(0))

        cps = []
        for g in range(_NG):
            cp = pltpu.make_async_copy(
                e_hbm.at[idx_v.at[buf, pl.ds(g * _GW, _GW)]],
                rows_v.at[buf, pl.ds(g * _GW, _GW)],
                gsem.at[buf],
            )
            cp.start()
            cps.append(cp)
        return cps

    def bias_add(buf):
        def bias_body(i, c):
            rows_v[buf, i] = rows_v[buf, i] + bias_sm[pl.ds(c, 16)]
            c = c + 16
            return jnp.where(c == _BIAS_PERIOD, 0, c)
        lax.fori_loop(0, _LK, bias_body, jnp.int32(0))

    def out_copy(t, buf):
        cps = []
        for b in range(_SB):
            cp = pltpu.make_async_copy(
                rows_v.at[buf, pl.ds(b * _KC, _KC)],
                out_hbm.at[batch0 + t * _SB + b],
                osem.at[buf],
            )
            cp.start()
            cps.append(cp)
        return cps

    def sub_chunk(t, carry):
        for cp in stage_and_fire(t, 0):
            cp.wait()
        bias_add(0)
        for cp in out_copy(t, 0):
            cp.wait()
        return carry

    lax.fori_loop(0, _NSUB, sub_chunk, jnp.int32(0))


def kernel(x, E, bc):
    mesh = plsc.VectorSubcoreMesh(
        core_axis_name="c", subcore_axis_name="s",
        num_cores=_NC, num_subcores=_NS)
    gather = pl.kernel(
        _sc_body,
        out_type=jax.ShapeDtypeStruct((_BATCH, _KC, _D), jnp.float32),
        mesh=mesh,
        scratch_types=[
            pltpu.VMEM((2, _LK), jnp.int32),
            pltpu.VMEM((2, _LK, _D), jnp.float32),
            pltpu.VMEM((_BIAS_PERIOD,), jnp.float32),
            pltpu.VMEM((_OFF_PERIOD,), jnp.int32),
            pltpu.SemaphoreType.DMA((2,)),
            pltpu.SemaphoreType.DMA((2,)),
        ],
        compiler_params=pltpu.CompilerParams(use_tc_tiling_on_sc=False),
    )
    return gather(
        x.reshape(-1).astype(jnp.int32),
        E,
        bc.reshape(-1),
        jnp.asarray(_OFF_PATTERN),
    )


# speedup vs baseline: 1.0105x; 1.0105x over previous
import jax
import jax.numpy as jnp
import numpy as np
from jax import lax
from jax.experimental import pallas as pl
from jax.experimental.pallas import tpu as pltpu
from jax.experimental.pallas import tpu_sc as plsc

_CAT_DIMS = [100000] * 26
_KC = len(_CAT_DIMS)
_D = 16
_BATCH = 16384
_NC, _NS = 2, 16
_NW = _NC * _NS
_BPW = _BATCH // _NW
_SB = 64
_NSUB = _BPW // _SB
_LK = _SB * _KC
_GW = 128
_NG = _LK // _GW
_OFF_PERIOD = 208
_BIAS_PERIOD = _KC * _D

_OFFSETS = np.cumsum([0] + _CAT_DIMS[:-1]).astype(np.int32)
_OFF_PATTERN = np.tile(_OFFSETS, _OFF_PERIOD // _KC)


def _sc_body(x_hbm, e_hbm, bias_hbm, off_hbm, out_hbm,
             idx_v, rows_v, bias_sm, off_sm, gsem, osem):
    wid = lax.axis_index("s") * _NC + lax.axis_index("c")
    pltpu.sync_copy(bias_hbm, bias_sm)
    pltpu.sync_copy(off_hbm, off_sm)
    base0 = wid * (_BPW * _KC)
    batch0 = wid * _BPW

    def stage_and_fire(t, buf):
        base = base0 + t * _LK
        pltpu.sync_copy(x_hbm.at[pl.ds(base, _LK)], idx_v.at[buf])

        def off_body(j, c):
            sl = pl.ds(j * 16, 16)
            idx_v[buf, sl] = idx_v[buf, sl] + off_sm[pl.ds(c, 16)]
            c = c + 16
            return jnp.where(c == _OFF_PERIOD, 0, c)
        lax.fori_loop(0, _LK // 16, off_body, jnp.int32(0))

        cps = []
        for g in range(_NG):
            cp = pltpu.make_async_copy(
                e_hbm.at[idx_v.at[buf, pl.ds(g * _GW, _GW)]],
                rows_v.at[buf, pl.ds(g * _GW, _GW)],
                gsem.at[buf],
            )
            cp.start()
            cps.append(cp)
        return cps

    def bias_add(buf):
        def bias_body(i, c):
            rows_v[buf, i] = rows_v[buf, i] + bias_sm[pl.ds(c, 16)]
            c = c + 16
            return jnp.where(c == _BIAS_PERIOD, 0, c)
        lax.fori_loop(0, _LK, bias_body, jnp.int32(0))

    def out_copy(t, buf):
        cps = []
        for b in range(_SB):
            cp = pltpu.make_async_copy(
                rows_v.at[buf, pl.ds(b * _KC, _KC)],
                out_hbm.at[batch0 + t * _SB + b],
                osem.at[buf],
            )
            cp.start()
            cps.append(cp)
        return cps

    gathers = {0: stage_and_fire(0, 0)}
    out_cps = {}
    for t in range(_NSUB):
        buf = t % 2
        for cp in gathers.pop(t):
            cp.wait()
        if t + 1 < _NSUB:
            nbuf = (t + 1) % 2
            if t - 1 in out_cps:
                for cp in out_cps.pop(t - 1):
                    cp.wait()
            gathers[t + 1] = stage_and_fire(t + 1, nbuf)
        bias_add(buf)
        out_cps[t] = out_copy(t, buf)
    for t in sorted(out_cps):
        for cp in out_cps.pop(t):
            cp.wait()


def kernel(x, E, bc):
    mesh = plsc.VectorSubcoreMesh(
        core_axis_name="c", subcore_axis_name="s",
        num_cores=_NC, num_subcores=_NS)
    gather = pl.kernel(
        _sc_body,
        out_type=jax.ShapeDtypeStruct((_BATCH, _KC, _D), jnp.float32),
        mesh=mesh,
        scratch_types=[
            pltpu.VMEM((2, _LK), jnp.int32),
            pltpu.VMEM((2, _LK, _D), jnp.float32),
            pltpu.VMEM((_BIAS_PERIOD,), jnp.float32),
            pltpu.VMEM((_OFF_PERIOD,), jnp.int32),
            pltpu.SemaphoreType.DMA((2,)),
            pltpu.SemaphoreType.DMA((2,)),
        ],
        compiler_params=pltpu.CompilerParams(use_tc_tiling_on_sc=False),
    )
    return gather(
        x.reshape(-1).astype(jnp.int32),
        E,
        bc.reshape(-1),
        jnp.asarray(_OFF_PATTERN),
    )
